# Initial kernel scaffold; baseline (speedup 1.0000x reference)
import jax
import jax.numpy as jnp
from jax.experimental import pallas as pl


def kernel(vertices, faces, faces_feature, face_edges, params):
    raise NotImplementedError("write your pallas kernel here")


# speedup vs baseline: 3.8727x; 3.8727x over previous
import functools

import jax
import jax.numpy as jnp
import numpy as np
from jax import lax
from jax.experimental import pallas as pl
from jax.experimental.pallas import tpu as pltpu
from jax.experimental.pallas import tpu_sc as plsc

_SC_PARAMS = dict(
    compiler_params=pltpu.CompilerParams(use_tc_tiling_on_sc=False))

NF = 10000
NFP = 10240
NV = 5000
E = 320000
NW = 32
EPW = 10000
EPWP = 10112
NB_E = 79
FPW = 320
NR = 10240
DUMP = 10000
ROWS_PER_SUB = 640


def _spl(v, dtype=jnp.int32):
    return jnp.full((16,), v, dtype)


def _fill_zero_rows(zbuf, nrows, w):
    zv = jnp.zeros((16,), jnp.float32)

    def body(r, _):
        for c in range(w // 16):
            zbuf[r, pl.ds(c * 16, 16)] = zv
        return 0

    lax.fori_loop(0, nrows, body, 0)


def _zero_acc(zbuf, acc, sid, w):
    def body(i, _):
        off = pl.multiple_of(sid * ROWS_PER_SUB + i * 64, 64)
        pltpu.sync_copy(zbuf, acc.at[pl.ds(off, 64)])
        return 0

    lax.fori_loop(0, ROWS_PER_SUB // 64, body, 0)


def _copy_out(acc, out_hbm, cid, sid, w):
    off = pl.multiple_of(sid * ROWS_PER_SUB, 64)
    pltpu.sync_copy(acc.at[pl.ds(off, ROWS_PER_SUB)],
                    out_hbm.at[cid].at[pl.ds(off, ROWS_PER_SUB)])


def _vgather_call(verts16, faces_idx):
    mesh = plsc.VectorSubcoreMesh(core_axis_name="c", subcore_axis_name="s")

    @functools.partial(
        pl.kernel, mesh=mesh, **_SC_PARAMS,
        out_type=jax.ShapeDtypeStruct((NW, 1024, 16), jnp.float32),
        scratch_types=[
            pltpu.VMEM((1024,), jnp.int32),
            pltpu.VMEM((128, 16), jnp.float32),
            pltpu.SemaphoreType.DMA,
        ],
    )
    def k(v_hbm, f_hbm, out_hbm, fidx, rows, sem):
        cid = lax.axis_index("c")
        sid = lax.axis_index("s")
        wid = cid * 16 + sid
        pltpu.sync_copy(f_hbm.at[wid], fidx)

        def body(b, _):
            off = pl.multiple_of(b * 128, 128)
            pltpu.async_copy(v_hbm.at[fidx.at[pl.ds(off, 128)]], rows, sem).wait()
            pltpu.sync_copy(rows, out_hbm.at[wid].at[pl.ds(off, 128)])
            return 0

        lax.fori_loop(0, 8, body, 0)

    return k(verts16, faces_idx)


def _quant_call(rows48, S):
    def body(vr, sr, disc_ref, idx_ref):
        t = (vr[...] + 1.0) * 64.0
        r = t.astype(jnp.int32)
        rf = r.astype(jnp.float32)
        tie = (rf == t) & ((r & 1) == 1)
        r = jnp.where(tie, r - 1, r)
        r = jnp.clip(r, 0, 127)
        d9 = _dot(r.astype(jnp.float32), sr[...])
        disc = d9.astype(jnp.int32)
        disc_ref[...] = disc
        j = lax.broadcasted_iota(jnp.int32, (_RB, 9), 1)
        idx_ref[...] = disc + 128 * j

    return pl.pallas_call(
        body,
        grid=(_NRB,),
        in_specs=[pl.BlockSpec((_RB, 48), lambda i: (i, 0)),
                  pl.BlockSpec((48, 9), lambda i: (0, 0))],
        out_specs=[pl.BlockSpec((_RB, 9), lambda i: (i, 0)),
                   pl.BlockSpec((_RB, 9), lambda i: (i, 0))],
        out_shape=[jax.ShapeDtypeStruct((NFP, 9), jnp.int32),
                   jax.ShapeDtypeStruct((NFP, 9), jnp.int32)],
    )(rows48, S)


def _embed_call(U, idx_w, dst_face, W):
    mesh = plsc.VectorSubcoreMesh(core_axis_name="c", subcore_axis_name="s")

    @functools.partial(
        pl.kernel, mesh=mesh, **_SC_PARAMS,
        out_type=jax.ShapeDtypeStruct((2, NR, W), jnp.float32),
        scratch_types=[
            pltpu.VMEM((23 * 128,), jnp.int32),
            pltpu.VMEM((23, 128), jnp.int32),
            pltpu.VMEM((128, W), jnp.float32),
            pltpu.VMEM((64, W), jnp.float32),
            pltpu.VMEM_SHARED((NR, W), jnp.float32),
            pltpu.SemaphoreType.DMA,
        ],
    )
    def k(u_hbm, i_hbm, df_hbm, out_hbm, gidx, didx, rows, zbuf, acc, sem):
        cid = lax.axis_index("c")
        sid = lax.axis_index("s")
        wid = cid * 16 + sid
        _fill_zero_rows(zbuf, 64, W)
        _zero_acc(zbuf, acc, sid, W)
        pltpu.sync_copy(i_hbm.at[wid], gidx)
        pltpu.sync_copy(df_hbm.at[wid], didx)
        plsc.subcore_barrier()

        def body(b, _):
            off = pl.multiple_of(b * 128, 128)
            pltpu.async_copy(u_hbm.at[gidx.at[pl.ds(off, 128)]], rows, sem).wait()
            pltpu.sync_copy(rows, acc.at[didx.at[b]], add=True)
            return 0

        lax.fori_loop(0, 23, body, 0)
        plsc.subcore_barrier()
        _copy_out(acc, out_hbm, cid, sid, W)

    return k(U, idx_w, dst_face)


def _seg_call(y, src_w, dst_w, w):
    mesh = plsc.VectorSubcoreMesh(core_axis_name="c", subcore_axis_name="s")

    @functools.partial(
        pl.kernel, mesh=mesh, **_SC_PARAMS,
        out_type=jax.ShapeDtypeStruct((2, NR, w), jnp.float32),
        scratch_types=[
            pltpu.VMEM((EPWP,), jnp.int32),
            pltpu.VMEM((NB_E, 128), jnp.int32),
            pltpu.VMEM((128, w), jnp.float32),
            pltpu.VMEM((64, w), jnp.float32),
            pltpu.VMEM_SHARED((NR, w), jnp.float32),
            pltpu.SemaphoreType.DMA,
        ],
    )
    def k(y_hbm, s_hbm, d_hbm, out_hbm, sidx, didx, rows, zbuf, acc, sem):
        cid = lax.axis_index("c")
        sid = lax.axis_index("s")
        wid = cid * 16 + sid
        _fill_zero_rows(zbuf, 64, w)
        _zero_acc(zbuf, acc, sid, w)
        pltpu.sync_copy(s_hbm.at[wid], sidx)
        pltpu.sync_copy(d_hbm.at[wid], didx)
        plsc.subcore_barrier()

        def body(b, _):
            off = pl.multiple_of(b * 128, 128)
            pltpu.async_copy(y_hbm.at[sidx.at[pl.ds(off, 128)]], rows, sem).wait()
            pltpu.sync_copy(rows, acc.at[didx.at[b]], add=True)
            return 0

        lax.fori_loop(0, NB_E, body, 0)
        plsc.subcore_barrier()
        _copy_out(acc, out_hbm, cid, sid, w)

    return k(y, src_w, dst_w)


def _dot(a, b):
    return jnp.dot(a, b, preferred_element_type=jnp.float32)


def _u_call(coor_embed, W9, b2):
    def body(ce, wj, bj, out):
        out[...] = (_dot(ce[...], wj[0]) + bj[...] * (1.0 / 9.0))[None]

    return pl.pallas_call(
        body,
        grid=(9,),
        in_specs=[
            pl.BlockSpec((128, 64), lambda j: (0, 0)),
            pl.BlockSpec((1, 64, 192), lambda j: (j, 0, 0)),
            pl.BlockSpec((1, 192), lambda j: (0, 0)),
        ],
        out_specs=pl.BlockSpec((1, 128, 192), lambda j: (j, 0, 0)),
        out_shape=jax.ShapeDtypeStruct((9, 128, 192), jnp.float32),
    )(coor_embed, W9, b2)


_RB = 1024
_NRB = NFP // _RB


def _rows_spec(w):
    return pl.BlockSpec((_RB, w), lambda i: (i, 0))


def _part_spec(w):
    return pl.BlockSpec((2, _RB, w), lambda i: (0, i, 0))


def _whole(shape):
    nd = len(shape)
    return pl.BlockSpec(shape, lambda i: (0,) * nd)


def _l2norm(o):
    n = jnp.sqrt(jnp.sum(o * o, axis=-1, keepdims=True))
    return o / jnp.maximum(n, 1e-12)


def _a0_call(pa, pb, Wpa, Wpb, bp, Wl):
    def body(par, pbr, wpa, wpb, b, wl, h_ref, y_ref):
        xa = par[0] + par[1]
        xb = pbr[0] + pbr[1]
        h = jax.nn.relu(_dot(xa, wpa[...]) + _dot(xb, wpb[...]) + b[...])
        h_ref[...] = h
        y_ref[:, 0:64] = _dot(h, wl[...])
        y_ref[:, 64:80] = jnp.ones((_RB, 16), jnp.float32)

    return pl.pallas_call(
        body,
        grid=(_NRB,),
        in_specs=[_part_spec(128), _part_spec(64), _whole((128, 192)),
                  _whole((64, 192)), _whole((1, 192)), _whole((192, 64))],
        out_specs=[_rows_spec(192), _rows_spec(80)],
        out_shape=[jax.ShapeDtypeStruct((NFP, 192), jnp.float32),
                   jax.ShapeDtypeStruct((NFP, 80), jnp.float32)],
    )(pa, pb, Wpa, Wpb, bp, Wl)


def _b0_call(p, h, Wr, bl, g, b):
    def body(pr, hr, wr, blr, gr, br, x_ref, ic_ref):
        q = pr[0] + pr[1]
        cnt = jnp.maximum(q[:, 64:65], 1.0)
        agg = q[:, 0:64] / cnt
        o = agg + blr[...] + _dot(hr[...], wr[...])
        o = _l2norm(o)
        o = o * jax.nn.sigmoid(o)
        mu = jnp.mean(o, axis=-1, keepdims=True)
        var = jnp.mean((o - mu) ** 2, axis=-1, keepdims=True)
        x_ref[...] = (o - mu) / jnp.sqrt(var + 1e-5) * gr[...] + br[...]
        ic_ref[...] = jnp.broadcast_to(1.0 / cnt, (_RB, 128))

    return pl.pallas_call(
        body,
        grid=(_NRB,),
        in_specs=[_part_spec(80), _rows_spec(192), _whole((192, 64)),
                  _whole((1, 64)), _whole((1, 64)), _whole((1, 64))],
        out_specs=[_rows_spec(64), _rows_spec(128)],
        out_shape=[jax.ShapeDtypeStruct((NFP, 64), jnp.float32),
                   jax.ShapeDtypeStruct((NFP, 128), jnp.float32)],
    )(p, h, Wr, bl, g, b)


def _a_call(x, Wp, bp, fi, split):
    def body(xr, wp, b, *outs):
        h = jax.nn.relu(_dot(xr[...], wp[...]) + b[...])
        if split:
            outs[0][...] = h[:, 0:128]
            outs[1][...] = h[:, 128:256]
        else:
            outs[0][...] = h

    if split:
        out_specs = [_rows_spec(128), _rows_spec(128)]
        out_shape = [jax.ShapeDtypeStruct((NFP, 128), jnp.float32)] * 2
    else:
        out_specs = _rows_spec(fi)
        out_shape = jax.ShapeDtypeStruct((NFP, fi), jnp.float32)
    return pl.pallas_call(
        body,
        grid=(_NRB,),
        in_specs=[_rows_spec(fi), _whole((fi, fi)), _whole((1, fi))],
        out_specs=out_specs,
        out_shape=out_shape,
    )(x, Wp, bp)


def _b_call(parts, hs, invc, Wls, bl, Wrs, fo):
    nk = len(parts)
    widths = [p.shape[-1] for p in parts]

    def body(*refs):
        prs = refs[:nk]
        hrs = refs[nk:2 * nk]
        icr = refs[2 * nk]
        wls = refs[2 * nk + 1:3 * nk + 1]
        blr = refs[3 * nk + 1]
        wrs = refs[3 * nk + 2:4 * nk + 2]
        out = refs[-1]
        ic = icr[...]
        o = jnp.broadcast_to(blr[...], (_RB, fo))
        for k in range(nk):
            q = prs[k][0] + prs[k][1]
            agg = q * ic[:, 0:widths[k]]
            o = o + _dot(agg, wls[k][...]) + _dot(hrs[k][...], wrs[k][...])
        out[...] = _l2norm(o)

    in_specs = ([_part_spec(w) for w in widths]
                + [_rows_spec(w) for w in widths]
                + [_rows_spec(128)]
                + [_whole((w, fo)) for w in widths]
                + [_whole((1, fo))]
                + [_whole((w, fo)) for w in widths])
    return pl.pallas_call(
        body,
        grid=(_NRB,),
        in_specs=in_specs,
        out_specs=_rows_spec(fo),
        out_shape=jax.ShapeDtypeStruct((NFP, fo), jnp.float32),
    )(*parts, *hs, invc, *Wls, bl, *Wrs)


def kernel(vertices, faces, faces_feature, face_edges, params):
    verts16 = jnp.pad(vertices.reshape(NV, 3), ((0, 0), (0, 13)))
    faces_pad = jnp.pad(faces.reshape(NF, 3), ((0, NFP - NF), (0, 0)))
    faces_idx = jnp.pad(faces_pad.reshape(NW, FPW * 3),
                        ((0, 0), (0, 1024 - FPW * 3)))

    fe = face_edges.reshape(E, 2)
    src = fe[:, 0].reshape(NW, EPW)
    dst = fe[:, 1].reshape(NW, EPW)
    src_w = jnp.pad(src, ((0, 0), (0, EPWP - EPW)))
    dst_w = jnp.pad(dst, ((0, 0), (0, EPWP - EPW)), constant_values=DUMP)
    dst_w = dst_w.reshape(NW, NB_E, 128)

    pos = jnp.arange(23 * 128, dtype=jnp.int32)
    wids = jnp.arange(NW, dtype=jnp.int32)[:, None]
    dst_face = jnp.where(pos[None, :] < FPW * 9,
                         wids * FPW + pos[None, :] // 9, DUMP)
    dst_face = dst_face.reshape(NW, 23, 128)

    vg = _vgather_call(verts16, faces_idx)
    rows48 = vg[:, :FPW * 3].reshape(NFP, 48)
    sel = np.zeros((48, 9), np.float32)
    for s in range(3):
        for c in range(3):
            sel[s * 16 + c, 3 * s + c] = 1.0
    disc, idxm = _quant_call(rows48, jnp.asarray(sel))
    disc_out = disc[:NF].reshape(1, NF, 9)
    idx_w = jnp.pad(idxm.reshape(NW, FPW * 9),
                    ((0, 0), (0, 2944 - FPW * 9)))

    p = params
    W9 = p['proj_in_W'].reshape(9, 64, 192)
    U = _u_call(p['coor_embed'], W9, p['proj_in_b'].reshape(1, 192))
    Uflat = U.reshape(9 * 128, 192)
    x0a = _embed_call(Uflat[:, :128], idx_w, dst_face, 128)
    x0b = _embed_call(Uflat[:, 128:], idx_w, dst_face, 64)

    convs = p['convs']
    c0, c1, c2, c3, c4 = convs

    h0, y80 = _a0_call(x0a, x0b, c0['Wp'][:128], c0['Wp'][128:],
                       c0['bp'].reshape(1, 192), c0['Wl'])
    p0 = _seg_call(y80, src_w, dst_w, 80)
    x1, invc = _b0_call(p0, h0, c0['Wr'], c0['bl'].reshape(1, 64),
                        p['ln_g'].reshape(1, 64), p['ln_b'].reshape(1, 64))

    h1 = _a_call(x1, c1['Wp'], c1['bp'].reshape(1, 64), 64, False)
    p1 = _seg_call(h1, src_w, dst_w, 64)
    x2 = _b_call([p1], [h1], invc, [c1['Wl']], c1['bl'].reshape(1, 128),
                 [c1['Wr']], 128)

    h2 = _a_call(x2, c2['Wp'], c2['bp'].reshape(1, 128), 128, False)
    p2 = _seg_call(h2, src_w, dst_w, 128)
    x3 = _b_call([p2], [h2], invc, [c2['Wl']], c2['bl'].reshape(1, 256),
                 [c2['Wr']], 256)

    h3a, h3b = _a_call(x3, c3['Wp'], c3['bp'].reshape(1, 256), 256, True)
    p3a = _seg_call(h3a, src_w, dst_w, 128)
    p3b = _seg_call(h3b, src_w, dst_w, 128)
    x4 = _b_call([p3a, p3b], [h3a, h3b], invc,
                 [c3['Wl'][:128], c3['Wl'][128:]], c3['bl'].reshape(1, 256),
                 [c3['Wr'][:128], c3['Wr'][128:]], 256)

    h4a, h4b = _a_call(x4, c4['Wp'], c4['bp'].reshape(1, 256), 256, True)
    p4a = _seg_call(h4a, src_w, dst_w, 128)
    p4b = _seg_call(h4b, src_w, dst_w, 128)
    x5 = _b_call([p4a, p4b], [h4a, h4b], invc,
                 [c4['Wl'][:128], c4['Wl'][128:]], c4['bl'].reshape(1, 576),
                 [c4['Wr'][:128], c4['Wr'][128:]], 576)

    out = x5[:NF].reshape(1, NF, 576)
    return out, disc_out
